# baseline (speedup 1.0000x reference)
import jax
import jax.numpy as jnp
from jax import lax
from jax.experimental.compute_on import compute_on
from jax.experimental import pallas as pl
from jax.experimental.pallas import tpu as pltpu
from jax.experimental.pallas import tpu_sc as plsc

N = 10000
E = 160000
D_IN = 512
D_H = 256
D_OUT = 128

_BLK = 1000

_NW = 32
_PER_W = 5120
_P = _NW * _PER_W
_K = 128
_NEXT = N + 16

_EB = 512
_NB = _P // _EB
_WIN = 256


def _mm_kernel(x_ref, w_ref, o_ref):
    o_ref[...] = jnp.dot(x_ref[...], w_ref[...],
                         preferred_element_type=jnp.float32)


def _matmul(x, w):
    m, k = x.shape
    n = w.shape[1]
    return pl.pallas_call(
        _mm_kernel,
        grid=(m // _BLK,),
        in_specs=[
            pl.BlockSpec((_BLK, k), lambda i: (i, 0)),
            pl.BlockSpec((k, n), lambda i: (0, 0)),
        ],
        out_specs=pl.BlockSpec((_BLK, n), lambda i: (i, 0)),
        out_shape=jax.ShapeDtypeStruct((m, n), jnp.float32),
        compiler_params=pltpu.CompilerParams(
            dimension_semantics=("parallel",)),
    )(x, w)


def _sc_gather_body(tab_hbm, idx_hbm, out_hbm, idx_v, buf0, buf1,
                    g0, g1, s0, s1):
    c = lax.axis_index("c")
    s = lax.axis_index("s")
    wid = s * 2 + c
    base = wid * _PER_W
    pltpu.sync_copy(idx_hbm.at[pl.ds(base, _PER_W)], idx_v)
    nch = _PER_W // _K

    def gather_slice(i, buf, gsem):
        return pltpu.make_async_copy(
            tab_hbm.at[idx_v.at[pl.ds(i * _K, _K)]], buf, gsem)

    def store_slice(i, buf, ssem):
        return pltpu.make_async_copy(
            buf, out_hbm.at[pl.ds(base + i * _K, _K)], ssem)

    gather_slice(0, buf0, g0).start()
    gather_slice(1, buf1, g1).start()

    @pl.loop(0, nch // 2)
    def _(ii):
        for par, (buf, gsem, ssem) in enumerate(
                ((buf0, g0, s0), (buf1, g1, s1))):
            i = ii * 2 + par
            gather_slice(i, buf, gsem).wait()
            st = store_slice(i, buf, ssem)
            st.start()

            @pl.when(i + 2 < nch)
            def _():
                st.wait()
                gather_slice(i + 2, buf, gsem).start()

    store_slice(nch - 2, buf0, s0).wait()
    store_slice(nch - 1, buf1, s1).wait()


def _sc_gather(table, idx):
    d = table.shape[1]
    mesh = plsc.VectorSubcoreMesh(core_axis_name="c", subcore_axis_name="s")
    fn = pl.kernel(
        _sc_gather_body,
        out_type=jax.ShapeDtypeStruct((_P, d), jnp.float32),
        mesh=mesh,
        scratch_types=[
            pltpu.VMEM((_PER_W,), jnp.int32),
            pltpu.VMEM((_K, d), jnp.float32),
            pltpu.VMEM((_K, d), jnp.float32),
            pltpu.SemaphoreType.DMA,
            pltpu.SemaphoreType.DMA,
            pltpu.SemaphoreType.DMA,
            pltpu.SemaphoreType.DMA,
        ],
    )
    return fn(table, idx)


def _sc_ea_body(p_hbm, q_hbm, row_hbm, col_hbm, out_hbm,
                p_v, q_v, r_v, c_v, ea_v):
    c = lax.axis_index("c")
    s = lax.axis_index("s")
    wid = s * 2 + c
    base = wid * _PER_W
    pltpu.sync_copy(p_hbm, p_v)
    pltpu.sync_copy(q_hbm, q_v)
    pltpu.sync_copy(row_hbm.at[pl.ds(base, _PER_W)], r_v)
    pltpu.sync_copy(col_hbm.at[pl.ds(base, _PER_W)], c_v)

    @pl.loop(0, _PER_W // 16)
    def _(i):
        ir = r_v[pl.ds(i * 16, 16)]
        ic = c_v[pl.ds(i * 16, 16)]
        pv = plsc.load_gather(p_v, [ir])
        qv = plsc.load_gather(q_v, [ic])
        ea_v[pl.ds(i * 16, 16)] = jnp.maximum(pv + qv, 0.0)

    pltpu.sync_copy(ea_v, out_hbm.at[pl.ds(base, _PER_W)])


def _sc_ea(p_ext, q_ext, row_s, col_s):
    mesh = plsc.VectorSubcoreMesh(core_axis_name="c", subcore_axis_name="s")
    fn = pl.kernel(
        _sc_ea_body,
        out_type=jax.ShapeDtypeStruct((_P,), jnp.float32),
        mesh=mesh,
        scratch_types=[
            pltpu.VMEM((_NEXT,), jnp.float32),
            pltpu.VMEM((_NEXT,), jnp.float32),
            pltpu.VMEM((_PER_W,), jnp.int32),
            pltpu.VMEM((_PER_W,), jnp.int32),
            pltpu.VMEM((_PER_W,), jnp.float32),
        ],
        compiler_params=pltpu.CompilerParams(needs_layout_passes=False),
    )
    return fn(p_ext, q_ext, row_s, col_s)


def _make_seg_kernel(edge_split):
    def _seg_kernel(c0_ref, m_ref, col_ref, val_ref, out_ref):
        h = pl.program_id(0)
        j = pl.program_id(1)

        @pl.when(j == 0)
        def _():
            out_ref[...] = jnp.zeros_like(out_ref)

        jj = j + h * pl.num_programs(1) if edge_split else j
        c0 = c0_ref[jj]
        loc = col_ref[0, 0, :] - c0 * 128
        vals = val_ref[0, 0, :]
        iota = lax.broadcasted_iota(jnp.int32, (_WIN, _EB), 0)
        onehot = (iota == loc[None, :]).astype(jnp.bfloat16)
        mv = m_ref[...] * vals[:, None]
        hi = mv.astype(jnp.bfloat16)
        lo = (mv - hi.astype(jnp.float32)).astype(jnp.bfloat16)
        prod = (jnp.dot(onehot, hi, preferred_element_type=jnp.float32)
                + jnp.dot(onehot, lo, preferred_element_type=jnp.float32))
        cur = out_ref[0, pl.ds(c0 * 128, _WIN), :]
        out_ref[0, pl.ds(c0 * 128, _WIN), :] = cur + prod

    return _seg_kernel


def _segment(c0, m, col3, val3, d):
    edge_split = d < 256
    dh = d if edge_split else d // 2
    nj = _NB // 2 if edge_split else _NB
    if edge_split:
        m_map = lambda h, j, s: (h * nj + j, 0)
        e_map = lambda h, j, s: (h * nj + j, 0, 0)
    else:
        m_map = lambda h, j, s: (j, h)
        e_map = lambda h, j, s: (j, 0, 0)
    grid_spec = pltpu.PrefetchScalarGridSpec(
        num_scalar_prefetch=1,
        grid=(2, nj),
        in_specs=[
            pl.BlockSpec((_EB, dh), m_map),
            pl.BlockSpec((1, 1, _EB), e_map),
            pl.BlockSpec((1, 1, _EB), e_map),
        ],
        out_specs=pl.BlockSpec((1, N + 240, dh), lambda h, j, s: (h, 0, 0)),
    )
    accs = pl.pallas_call(
        _make_seg_kernel(edge_split),
        grid_spec=grid_spec,
        out_shape=jax.ShapeDtypeStruct((2, N + 240, dh), jnp.float32),
        compiler_params=pltpu.CompilerParams(
            dimension_semantics=("parallel", "arbitrary")),
    )(c0, m, col3, val3)
    if edge_split:
        return accs[0] + accs[1]
    return jnp.transpose(accs, (1, 0, 2)).reshape(N + 240, d)


def kernel(node_attr, edge_attr, edge_index, W1, b1, W2, b2, Wm, bm):
    row = edge_index[0]
    col = edge_index[1]
    ew = edge_attr.reshape(-1)

    npad = _P - E
    colp = jnp.concatenate([col, jnp.full((npad,), N - 1, jnp.int32)])
    rowp = jnp.concatenate(
        [row, N + (jnp.arange(npad, dtype=jnp.int32) % 16)])
    ewp = jnp.concatenate([ew, jnp.zeros((npad,), jnp.float32)])
    @compute_on("tpu_sparsecore")
    @jax.jit
    def _sc_sort(c, r, w):
        return lax.sort((c, r, w), num_keys=1)

    col_s, row_s, ew_s = _sc_sort(colp, rowp, ewp)

    c0 = col_s[::_EB] // 128
    col3 = col_s.reshape(_NB, 1, _EB)
    pad16 = ((0, 16), (0, 0))

    deg1 = jnp.ones((N,), jnp.float32).at[col].add(ew)
    dinv1 = lax.rsqrt(deg1)
    xw1 = _matmul(node_attr, W1)
    z1 = xw1 * dinv1[:, None]
    m1 = _sc_gather(jnp.pad(z1, pad16), row_s)
    acc1 = _segment(c0, m1, col3, ew_s.reshape(_NB, 1, _EB), D_H)
    h = jax.nn.relu(dinv1[:, None] * (acc1[:N] + z1) + b1)

    wm_a = Wm[:D_H, :]
    wm_b = Wm[D_H:, :]
    wext = jnp.concatenate([W2, wm_a, wm_b], axis=1)
    wext = jnp.pad(wext, ((0, 0), (0, 126)))
    big = _matmul(h, wext)
    hw2 = big[:, :D_OUT]
    p = big[:, D_OUT]
    q = big[:, D_OUT + 1]

    p_ext = jnp.concatenate([p + bm[0], jnp.full((16,), -1e30, jnp.float32)])
    q_ext = jnp.concatenate([q, jnp.zeros((16,), jnp.float32)])
    ea_s = _sc_ea(p_ext, q_ext, row_s, col_s)

    deg2 = jnp.ones((N,), jnp.float32).at[col_s].add(
        ea_s, indices_are_sorted=True)
    dinv2 = lax.rsqrt(deg2)
    z2 = hw2 * dinv2[:, None]
    m2 = _sc_gather(jnp.pad(z2, pad16), row_s)
    acc2 = _segment(c0, m2, col3, ea_s.reshape(_NB, 1, _EB), D_OUT)
    out = dinv2[:, None] * (acc2[:N] + z2) + b2
    return out


# speedup vs baseline: 1.1075x; 1.1075x over previous
import jax
import jax.numpy as jnp
from jax import lax
from jax.experimental import pallas as pl
from jax.experimental.pallas import tpu as pltpu
from jax.experimental.pallas import tpu_sc as plsc

N = 10000
E = 160000
D_IN = 512
D_H = 256
D_OUT = 128

_BLK = 1000

_NW = 32
_PER_W = 5120
_P = _NW * _PER_W
_K = 128
_NEXT = N + 16

_EB = 512
_NB = _P // _EB
_WIN = 256


def _mm_kernel(x_ref, w_ref, o_ref):
    o_ref[...] = jnp.dot(x_ref[...], w_ref[...],
                         preferred_element_type=jnp.float32)


def _matmul(x, w):
    m, k = x.shape
    n = w.shape[1]
    return pl.pallas_call(
        _mm_kernel,
        grid=(m // _BLK,),
        in_specs=[
            pl.BlockSpec((_BLK, k), lambda i: (i, 0)),
            pl.BlockSpec((k, n), lambda i: (0, 0)),
        ],
        out_specs=pl.BlockSpec((_BLK, n), lambda i: (i, 0)),
        out_shape=jax.ShapeDtypeStruct((m, n), jnp.float32),
        compiler_params=pltpu.CompilerParams(
            dimension_semantics=("parallel",)),
    )(x, w)


def _sc_gather_body(tab_hbm, idx_hbm, out_hbm, idx_v, buf0, buf1,
                    g0, g1, s0, s1):
    c = lax.axis_index("c")
    s = lax.axis_index("s")
    wid = s * 2 + c
    base = wid * _PER_W
    pltpu.sync_copy(idx_hbm.at[pl.ds(base, _PER_W)], idx_v)
    nch = _PER_W // _K

    def gather_slice(i, buf, gsem):
        return pltpu.make_async_copy(
            tab_hbm.at[idx_v.at[pl.ds(i * _K, _K)]], buf, gsem)

    def store_slice(i, buf, ssem):
        return pltpu.make_async_copy(
            buf, out_hbm.at[pl.ds(base + i * _K, _K)], ssem)

    gather_slice(0, buf0, g0).start()
    gather_slice(1, buf1, g1).start()

    @pl.loop(0, nch // 2)
    def _(ii):
        for par, (buf, gsem, ssem) in enumerate(
                ((buf0, g0, s0), (buf1, g1, s1))):
            i = ii * 2 + par
            gather_slice(i, buf, gsem).wait()
            st = store_slice(i, buf, ssem)
            st.start()

            @pl.when(i + 2 < nch)
            def _():
                st.wait()
                gather_slice(i + 2, buf, gsem).start()

    store_slice(nch - 2, buf0, s0).wait()
    store_slice(nch - 1, buf1, s1).wait()


def _sc_gather(table, idx):
    d = table.shape[1]
    mesh = plsc.VectorSubcoreMesh(core_axis_name="c", subcore_axis_name="s")
    fn = pl.kernel(
        _sc_gather_body,
        out_type=jax.ShapeDtypeStruct((_P, d), table.dtype),
        mesh=mesh,
        scratch_types=[
            pltpu.VMEM((_PER_W,), jnp.int32),
            pltpu.VMEM((_K, d), table.dtype),
            pltpu.VMEM((_K, d), table.dtype),
            pltpu.SemaphoreType.DMA,
            pltpu.SemaphoreType.DMA,
            pltpu.SemaphoreType.DMA,
            pltpu.SemaphoreType.DMA,
        ],
    )
    return fn(table, idx)


_KE = 512


def _sc_pair_body(a_hbm, b_hbm, idx_hbm, ao_hbm, bo_hbm,
                  idx_v, a0, a1, b0, b1, ga0, ga1, gb0, gb1,
                  sa0, sa1, sb0, sb1):
    c = lax.axis_index("c")
    s = lax.axis_index("s")
    wid = s * 2 + c
    base = wid * _PER_W
    pltpu.sync_copy(idx_hbm.at[pl.ds(base, _PER_W)], idx_v)
    nch = _PER_W // _KE

    def gathers(i, abuf, bbuf, gsa, gsb):
        sl = idx_v.at[pl.ds(i * _KE, _KE)]
        return (pltpu.make_async_copy(a_hbm.at[sl], abuf, gsa),
                pltpu.make_async_copy(b_hbm.at[sl], bbuf, gsb))

    def stores(i, abuf, bbuf, ssa, ssb):
        return (pltpu.make_async_copy(abuf, ao_hbm.at[pl.ds(base + i * _KE, _KE)], ssa),
                pltpu.make_async_copy(bbuf, bo_hbm.at[pl.ds(base + i * _KE, _KE)], ssb))

    for g in gathers(0, a0, b0, ga0, gb0):
        g.start()
    for g in gathers(1, a1, b1, ga1, gb1):
        g.start()

    @pl.loop(0, nch // 2)
    def _(ii):
        for par, (abuf, bbuf, gsa, gsb, ssa, ssb) in enumerate(
                ((a0, b0, ga0, gb0, sa0, sb0), (a1, b1, ga1, gb1, sa1, sb1))):
            i = ii * 2 + par
            for g in gathers(i, abuf, bbuf, gsa, gsb):
                g.wait()
            sts = stores(i, abuf, bbuf, ssa, ssb)
            for st in sts:
                st.start()

            @pl.when(i + 2 < nch)
            def _():
                for st in sts:
                    st.wait()
                for g in gathers(i + 2, abuf, bbuf, gsa, gsb):
                    g.start()

    for st in stores(nch - 2, a0, b0, sa0, sb0):
        st.wait()
    for st in stores(nch - 1, a1, b1, sa1, sb1):
        st.wait()


def _sc_pair_gather(a, b, idx):
    mesh = plsc.VectorSubcoreMesh(core_axis_name="c", subcore_axis_name="s")
    fn = pl.kernel(
        _sc_pair_body,
        out_type=(jax.ShapeDtypeStruct((_P,), a.dtype),
                  jax.ShapeDtypeStruct((_P,), b.dtype)),
        mesh=mesh,
        scratch_types=(
            [pltpu.VMEM((_PER_W,), jnp.int32)]
            + [pltpu.VMEM((_KE,), a.dtype) for _ in range(2)]
            + [pltpu.VMEM((_KE,), b.dtype) for _ in range(2)]
            + [pltpu.SemaphoreType.DMA for _ in range(8)]
        ),
    )
    return fn(a, b, idx)


def _sc_ea_body(p_hbm, q_hbm, row_hbm, col_hbm, out_hbm,
                p_v, q_v, r_v, c_v, ea_v):
    c = lax.axis_index("c")
    s = lax.axis_index("s")
    wid = s * 2 + c
    base = wid * _PER_W
    pltpu.sync_copy(p_hbm, p_v)
    pltpu.sync_copy(q_hbm, q_v)
    pltpu.sync_copy(row_hbm.at[pl.ds(base, _PER_W)], r_v)
    pltpu.sync_copy(col_hbm.at[pl.ds(base, _PER_W)], c_v)

    @pl.loop(0, _PER_W // 16)
    def _(i):
        ir = r_v[pl.ds(i * 16, 16)]
        ic = c_v[pl.ds(i * 16, 16)]
        pv = plsc.load_gather(p_v, [ir])
        qv = plsc.load_gather(q_v, [ic])
        ea_v[pl.ds(i * 16, 16)] = jnp.maximum(pv + qv, 0.0)

    pltpu.sync_copy(ea_v, out_hbm.at[pl.ds(base, _PER_W)])


def _sc_ea(p_ext, q_ext, row_s, col_s):
    mesh = plsc.VectorSubcoreMesh(core_axis_name="c", subcore_axis_name="s")
    fn = pl.kernel(
        _sc_ea_body,
        out_type=jax.ShapeDtypeStruct((_P,), jnp.float32),
        mesh=mesh,
        scratch_types=[
            pltpu.VMEM((_NEXT,), jnp.float32),
            pltpu.VMEM((_NEXT,), jnp.float32),
            pltpu.VMEM((_PER_W,), jnp.int32),
            pltpu.VMEM((_PER_W,), jnp.int32),
            pltpu.VMEM((_PER_W,), jnp.float32),
        ],
        compiler_params=pltpu.CompilerParams(needs_layout_passes=False),
    )
    return fn(p_ext, q_ext, row_s, col_s)


def _make_seg_kernel(edge_split):
    def _seg_kernel(c0_ref, m_ref, col_ref, val_ref, out_ref):
        h = pl.program_id(0)
        j = pl.program_id(1)

        @pl.when(j == 0)
        def _():
            out_ref[...] = jnp.zeros_like(out_ref)

        jj = j + h * pl.num_programs(1) if edge_split else j
        c0 = c0_ref[jj]
        loc = col_ref[0, 0, :] - c0 * 128
        vals = val_ref[0, 0, :]
        iota = lax.broadcasted_iota(jnp.int32, (_WIN, _EB), 0)
        onehot = (iota == loc[None, :]).astype(jnp.bfloat16)
        mv = m_ref[...] * vals[:, None]
        hi = mv.astype(jnp.bfloat16)
        lo = (mv - hi.astype(jnp.float32)).astype(jnp.bfloat16)
        prod = (jnp.dot(onehot, hi, preferred_element_type=jnp.float32)
                + jnp.dot(onehot, lo, preferred_element_type=jnp.float32))
        cur = out_ref[0, pl.ds(c0 * 128, _WIN), :]
        out_ref[0, pl.ds(c0 * 128, _WIN), :] = cur + prod

    return _seg_kernel


def _segment(c0, m, col3, val3, d):
    edge_split = d < 256
    dh = d if edge_split else d // 2
    nj = _NB // 2 if edge_split else _NB
    if edge_split:
        m_map = lambda h, j, s: (h * nj + j, 0)
        e_map = lambda h, j, s: (h * nj + j, 0, 0)
    else:
        m_map = lambda h, j, s: (j, h)
        e_map = lambda h, j, s: (j, 0, 0)
    grid_spec = pltpu.PrefetchScalarGridSpec(
        num_scalar_prefetch=1,
        grid=(2, nj),
        in_specs=[
            pl.BlockSpec((_EB, dh), m_map),
            pl.BlockSpec((1, 1, _EB), e_map),
            pl.BlockSpec((1, 1, _EB), e_map),
        ],
        out_specs=pl.BlockSpec((1, N + 240, dh), lambda h, j, s: (h, 0, 0)),
    )
    accs = pl.pallas_call(
        _make_seg_kernel(edge_split),
        grid_spec=grid_spec,
        out_shape=jax.ShapeDtypeStruct((2, N + 240, dh), jnp.float32),
        compiler_params=pltpu.CompilerParams(
            dimension_semantics=("parallel", "arbitrary")),
    )(c0, m, col3, val3)
    if edge_split:
        return accs[0] + accs[1]
    return jnp.transpose(accs, (1, 0, 2)).reshape(N + 240, d)


def kernel(node_attr, edge_attr, edge_index, W1, b1, W2, b2, Wm, bm):
    row = edge_index[0]
    col = edge_index[1]
    ew = edge_attr.reshape(-1)

    npad = _P - E
    colp = jnp.concatenate([col, jnp.full((npad,), N - 1, jnp.int32)])
    rowp = jnp.concatenate(
        [row, N + (jnp.arange(npad, dtype=jnp.int32) % 16)])
    ewp = jnp.concatenate([ew, jnp.zeros((npad,), jnp.float32)])

    packed = (colp.astype(jnp.uint32) << 18) | jnp.arange(_P, dtype=jnp.uint32)
    packed_s = lax.sort(packed)
    col_s = (packed_s >> 18).astype(jnp.int32)
    id_s = (packed_s & 0x3FFFF).astype(jnp.int32)
    row_s, ew_s = _sc_pair_gather(rowp, ewp, id_s)

    c0 = col_s[::_EB] // 128
    col3 = col_s.reshape(_NB, 1, _EB)
    pad16 = ((0, 16), (0, 0))

    deg1 = jnp.ones((N,), jnp.float32).at[col_s].add(
        ew_s, indices_are_sorted=True)
    dinv1 = lax.rsqrt(deg1)
    xw1 = _matmul(node_attr, W1)
    z1 = xw1 * dinv1[:, None]
    m1 = _sc_gather(jnp.pad(z1, pad16), row_s)
    acc1 = _segment(c0, m1, col3, ew_s.reshape(_NB, 1, _EB), D_H)
    h = jax.nn.relu(dinv1[:, None] * (acc1[:N] + z1) + b1)

    wm_a = Wm[:D_H, :]
    wm_b = Wm[D_H:, :]
    wext = jnp.concatenate([W2, wm_a, wm_b], axis=1)
    wext = jnp.pad(wext, ((0, 0), (0, 126)))
    big = _matmul(h, wext)
    hw2 = big[:, :D_OUT]
    p = big[:, D_OUT]
    q = big[:, D_OUT + 1]

    p_ext = jnp.concatenate([p + bm[0], jnp.full((16,), -1e30, jnp.float32)])
    q_ext = jnp.concatenate([q, jnp.zeros((16,), jnp.float32)])
    ea_s = _sc_ea(p_ext, q_ext, row_s, col_s)

    deg2 = jnp.ones((N,), jnp.float32).at[col_s].add(
        ea_s, indices_are_sorted=True)
    dinv2 = lax.rsqrt(deg2)
    z2 = hw2 * dinv2[:, None]
    m2 = _sc_gather(jnp.pad(z2, pad16), row_s)
    acc2 = _segment(c0, m2, col3, ea_s.reshape(_NB, 1, _EB), D_OUT)
    out = dinv2[:, None] * (acc2[:N] + z2) + b2
    return out


# speedup vs baseline: 1.3034x; 1.1768x over previous
import jax
import jax.numpy as jnp
from jax import lax
from jax.experimental import pallas as pl
from jax.experimental.pallas import tpu as pltpu
from jax.experimental.pallas import tpu_sc as plsc

N = 10000
E = 160000
D_IN = 512
D_H = 256
D_OUT = 128

_BLK = 1000

_NW = 32
_PER_W = 5120
_P = _NW * _PER_W
_K = 128
_NEXT = N + 16

_EB = 512
_NB = _P // _EB
_WIN = 256


def _mm_kernel(x_ref, w_ref, o_ref):
    o_ref[...] = jnp.dot(x_ref[...], w_ref[...],
                         preferred_element_type=jnp.float32)


def _matmul(x, w):
    m, k = x.shape
    n = w.shape[1]
    return pl.pallas_call(
        _mm_kernel,
        grid=(m // _BLK,),
        in_specs=[
            pl.BlockSpec((_BLK, k), lambda i: (i, 0)),
            pl.BlockSpec((k, n), lambda i: (0, 0)),
        ],
        out_specs=pl.BlockSpec((_BLK, n), lambda i: (i, 0)),
        out_shape=jax.ShapeDtypeStruct((m, n), jnp.float32),
        compiler_params=pltpu.CompilerParams(
            dimension_semantics=("parallel",)),
    )(x, w)


def _sc_gather_body(tab_hbm, idx_hbm, out_hbm, idx_v, buf0, buf1,
                    g0, g1, s0, s1):
    c = lax.axis_index("c")
    s = lax.axis_index("s")
    wid = s * 2 + c
    base = wid * _PER_W
    pltpu.sync_copy(idx_hbm.at[pl.ds(base, _PER_W)], idx_v)
    nch = _PER_W // _K

    def gather_slice(i, buf, gsem):
        return pltpu.make_async_copy(
            tab_hbm.at[idx_v.at[pl.ds(i * _K, _K)]], buf, gsem)

    def store_slice(i, buf, ssem):
        return pltpu.make_async_copy(
            buf, out_hbm.at[pl.ds(base + i * _K, _K)], ssem)

    gather_slice(0, buf0, g0).start()
    gather_slice(1, buf1, g1).start()

    @pl.loop(0, nch // 2)
    def _(ii):
        for par, (buf, gsem, ssem) in enumerate(
                ((buf0, g0, s0), (buf1, g1, s1))):
            i = ii * 2 + par
            gather_slice(i, buf, gsem).wait()
            st = store_slice(i, buf, ssem)
            st.start()

            @pl.when(i + 2 < nch)
            def _():
                st.wait()
                gather_slice(i + 2, buf, gsem).start()

    store_slice(nch - 2, buf0, s0).wait()
    store_slice(nch - 1, buf1, s1).wait()


def _sc_gather(table, idx):
    d = table.shape[1]
    mesh = plsc.VectorSubcoreMesh(core_axis_name="c", subcore_axis_name="s")
    fn = pl.kernel(
        _sc_gather_body,
        out_type=jax.ShapeDtypeStruct((_P, d), table.dtype),
        mesh=mesh,
        scratch_types=[
            pltpu.VMEM((_PER_W,), jnp.int32),
            pltpu.VMEM((_K, d), table.dtype),
            pltpu.VMEM((_K, d), table.dtype),
            pltpu.SemaphoreType.DMA,
            pltpu.SemaphoreType.DMA,
            pltpu.SemaphoreType.DMA,
            pltpu.SemaphoreType.DMA,
        ],
    )
    return fn(table, idx)


_KE = 512


def _sc_pair_body(a_hbm, b_hbm, idx_hbm, ao_hbm, bo_hbm,
                  idx_v, a0, a1, b0, b1, ga0, ga1, gb0, gb1,
                  sa0, sa1, sb0, sb1):
    c = lax.axis_index("c")
    s = lax.axis_index("s")
    wid = s * 2 + c
    base = wid * _PER_W
    pltpu.sync_copy(idx_hbm.at[pl.ds(base, _PER_W)], idx_v)
    nch = _PER_W // _KE

    def gathers(i, abuf, bbuf, gsa, gsb):
        sl = idx_v.at[pl.ds(i * _KE, _KE)]
        return (pltpu.make_async_copy(a_hbm.at[sl], abuf, gsa),
                pltpu.make_async_copy(b_hbm.at[sl], bbuf, gsb))

    def stores(i, abuf, bbuf, ssa, ssb):
        return (pltpu.make_async_copy(abuf, ao_hbm.at[pl.ds(base + i * _KE, _KE)], ssa),
                pltpu.make_async_copy(bbuf, bo_hbm.at[pl.ds(base + i * _KE, _KE)], ssb))

    for g in gathers(0, a0, b0, ga0, gb0):
        g.start()
    for g in gathers(1, a1, b1, ga1, gb1):
        g.start()

    @pl.loop(0, nch // 2)
    def _(ii):
        for par, (abuf, bbuf, gsa, gsb, ssa, ssb) in enumerate(
                ((a0, b0, ga0, gb0, sa0, sb0), (a1, b1, ga1, gb1, sa1, sb1))):
            i = ii * 2 + par
            for g in gathers(i, abuf, bbuf, gsa, gsb):
                g.wait()
            sts = stores(i, abuf, bbuf, ssa, ssb)
            for st in sts:
                st.start()

            @pl.when(i + 2 < nch)
            def _():
                for st in sts:
                    st.wait()
                for g in gathers(i + 2, abuf, bbuf, gsa, gsb):
                    g.start()

    for st in stores(nch - 2, a0, b0, sa0, sb0):
        st.wait()
    for st in stores(nch - 1, a1, b1, sa1, sb1):
        st.wait()


def _sc_pair_gather(a, b, idx):
    mesh = plsc.VectorSubcoreMesh(core_axis_name="c", subcore_axis_name="s")
    fn = pl.kernel(
        _sc_pair_body,
        out_type=(jax.ShapeDtypeStruct((_P,), a.dtype),
                  jax.ShapeDtypeStruct((_P,), b.dtype)),
        mesh=mesh,
        scratch_types=(
            [pltpu.VMEM((_PER_W,), jnp.int32)]
            + [pltpu.VMEM((_KE,), a.dtype) for _ in range(2)]
            + [pltpu.VMEM((_KE,), b.dtype) for _ in range(2)]
            + [pltpu.SemaphoreType.DMA for _ in range(8)]
        ),
    )
    return fn(a, b, idx)


def _sc_ea_body(p_hbm, q_hbm, row_hbm, col_hbm, out_hbm,
                p_v, q_v, r_v, c_v, ea_v):
    c = lax.axis_index("c")
    s = lax.axis_index("s")
    wid = s * 2 + c
    base = wid * _PER_W
    pltpu.sync_copy(p_hbm, p_v)
    pltpu.sync_copy(q_hbm, q_v)
    pltpu.sync_copy(row_hbm.at[pl.ds(base, _PER_W)], r_v)
    pltpu.sync_copy(col_hbm.at[pl.ds(base, _PER_W)], c_v)

    @pl.loop(0, _PER_W // 16)
    def _(i):
        ir = r_v[pl.ds(i * 16, 16)]
        ic = c_v[pl.ds(i * 16, 16)]
        pv = plsc.load_gather(p_v, [ir])
        qv = plsc.load_gather(q_v, [ic])
        ea_v[pl.ds(i * 16, 16)] = jnp.maximum(pv + qv, 0.0)

    pltpu.sync_copy(ea_v, out_hbm.at[pl.ds(base, _PER_W)])


def _sc_ea(p_ext, q_ext, row_s, col_s):
    mesh = plsc.VectorSubcoreMesh(core_axis_name="c", subcore_axis_name="s")
    fn = pl.kernel(
        _sc_ea_body,
        out_type=jax.ShapeDtypeStruct((_P,), jnp.float32),
        mesh=mesh,
        scratch_types=[
            pltpu.VMEM((_NEXT,), jnp.float32),
            pltpu.VMEM((_NEXT,), jnp.float32),
            pltpu.VMEM((_PER_W,), jnp.int32),
            pltpu.VMEM((_PER_W,), jnp.int32),
            pltpu.VMEM((_PER_W,), jnp.float32),
        ],
        compiler_params=pltpu.CompilerParams(needs_layout_passes=False),
    )
    return fn(p_ext, q_ext, row_s, col_s)


def _make_seg_kernel(edge_split):
    def _seg_kernel(c0_ref, m_ref, col_ref, val_ref, out_ref):
        h = pl.program_id(0)
        j = pl.program_id(1)

        @pl.when(j == 0)
        def _():
            out_ref[...] = jnp.zeros_like(out_ref)

        jj = j + h * pl.num_programs(1) if edge_split else j
        c0 = c0_ref[jj]
        loc = col_ref[0, 0, :] - c0 * 128
        vals = val_ref[0, 0, :]
        iota = lax.broadcasted_iota(jnp.int32, (_WIN, _EB), 0)
        onehot = (iota == loc[None, :]).astype(jnp.bfloat16)
        mv = m_ref[...] * vals[:, None]
        hi = mv.astype(jnp.bfloat16)
        lo = (mv - hi.astype(jnp.float32)).astype(jnp.bfloat16)
        prod = (jnp.dot(onehot, hi, preferred_element_type=jnp.float32)
                + jnp.dot(onehot, lo, preferred_element_type=jnp.float32))
        cur = out_ref[0, pl.ds(c0 * 128, _WIN), :]
        out_ref[0, pl.ds(c0 * 128, _WIN), :] = cur + prod

    return _seg_kernel


def _segment(c0, m, col3, val3, d):
    edge_split = True
    dh = d if edge_split else d // 2
    nj = _NB // 2 if edge_split else _NB
    if edge_split:
        m_map = lambda h, j, s: (h * nj + j, 0)
        e_map = lambda h, j, s: (h * nj + j, 0, 0)
    else:
        m_map = lambda h, j, s: (j, h)
        e_map = lambda h, j, s: (j, 0, 0)
    grid_spec = pltpu.PrefetchScalarGridSpec(
        num_scalar_prefetch=1,
        grid=(2, nj),
        in_specs=[
            pl.BlockSpec((_EB, dh), m_map),
            pl.BlockSpec((1, 1, _EB), e_map),
            pl.BlockSpec((1, 1, _EB), e_map),
        ],
        out_specs=pl.BlockSpec((1, N + 240, dh), lambda h, j, s: (h, 0, 0)),
    )
    accs = pl.pallas_call(
        _make_seg_kernel(edge_split),
        grid_spec=grid_spec,
        out_shape=jax.ShapeDtypeStruct((2, N + 240, dh), jnp.float32),
        compiler_params=pltpu.CompilerParams(
            dimension_semantics=("parallel", "arbitrary")),
    )(c0, m, col3, val3)
    if edge_split:
        return accs[0] + accs[1]
    return jnp.transpose(accs, (1, 0, 2)).reshape(N + 240, d)


def kernel(node_attr, edge_attr, edge_index, W1, b1, W2, b2, Wm, bm):
    row = edge_index[0]
    col = edge_index[1]
    ew = edge_attr.reshape(-1)

    npad = _P - E
    colp = jnp.concatenate([col, jnp.full((npad,), N - 1, jnp.int32)])
    rowp = jnp.concatenate(
        [row, N + (jnp.arange(npad, dtype=jnp.int32) % 16)])
    ewp = jnp.concatenate([ew, jnp.zeros((npad,), jnp.float32)])

    packed = (colp.astype(jnp.uint32) << 18) | jnp.arange(_P, dtype=jnp.uint32)
    packed_s = lax.sort(packed)
    col_s = (packed_s >> 18).astype(jnp.int32)
    id_s = (packed_s & 0x3FFFF).astype(jnp.int32)
    row_s, ew_s = _sc_pair_gather(rowp, ewp, id_s)

    c0 = col_s[::_EB] // 128
    col3 = col_s.reshape(_NB, 1, _EB)
    pad16 = ((0, 16), (0, 0))

    deg1 = jnp.ones((N,), jnp.float32).at[col_s].add(
        ew_s, indices_are_sorted=True)
    dinv1 = lax.rsqrt(deg1)
    xw1 = _matmul(node_attr, W1)
    z1 = xw1 * dinv1[:, None]
    m1 = _sc_gather(jnp.pad(z1, pad16), row_s)
    acc1 = _segment(c0, m1, col3, ew_s.reshape(_NB, 1, _EB), D_H)
    h = jax.nn.relu(dinv1[:, None] * (acc1[:N] + z1) + b1)

    wm_a = Wm[:D_H, :]
    wm_b = Wm[D_H:, :]
    wext = jnp.concatenate([W2, wm_a, wm_b], axis=1)
    wext = jnp.pad(wext, ((0, 0), (0, 126)))
    big = _matmul(h, wext)
    hw2 = big[:, :D_OUT]
    p = big[:, D_OUT]
    q = big[:, D_OUT + 1]

    p_ext = jnp.concatenate([p + bm[0], jnp.full((16,), -1e30, jnp.float32)])
    q_ext = jnp.concatenate([q, jnp.zeros((16,), jnp.float32)])
    ea_s = _sc_ea(p_ext, q_ext, row_s, col_s)

    deg2 = jnp.ones((N,), jnp.float32).at[col_s].add(
        ea_s, indices_are_sorted=True)
    dinv2 = lax.rsqrt(deg2)
    z2 = hw2 * dinv2[:, None]
    m2 = _sc_gather(jnp.pad(z2, pad16), row_s)
    acc2 = _segment(c0, m2, col3, ea_s.reshape(_NB, 1, _EB), D_OUT)
    out = dinv2[:, None] * (acc2[:N] + z2) + b2
    return out


# speedup vs baseline: 1.5611x; 1.1978x over previous
import jax
import jax.numpy as jnp
from jax import lax
from jax.experimental import pallas as pl
from jax.experimental.pallas import tpu as pltpu
from jax.experimental.pallas import tpu_sc as plsc

N = 10000
E = 160000
D_IN = 512
D_H = 256
D_OUT = 128

_BLK = 1000

_NW = 32
_PER_W = 5120
_P = _NW * _PER_W
_K = 128
_NEXT = N + 16

_EB = 1024
_NB = _P // _EB
_WIN = 256


def _mm_kernel(x_ref, w_ref, o_ref):
    o_ref[...] = jnp.dot(x_ref[...], w_ref[...],
                         preferred_element_type=jnp.float32)


def _matmul(x, w):
    m, k = x.shape
    n = w.shape[1]
    return pl.pallas_call(
        _mm_kernel,
        grid=(m // _BLK,),
        in_specs=[
            pl.BlockSpec((_BLK, k), lambda i: (i, 0)),
            pl.BlockSpec((k, n), lambda i: (0, 0)),
        ],
        out_specs=pl.BlockSpec((_BLK, n), lambda i: (i, 0)),
        out_shape=jax.ShapeDtypeStruct((m, n), jnp.float32),
        compiler_params=pltpu.CompilerParams(
            dimension_semantics=("parallel",)),
    )(x, w)


def _sc_gather_body(tab_hbm, idx_hbm, out_hbm, idx_v, buf0, buf1,
                    g0, g1, s0, s1):
    c = lax.axis_index("c")
    s = lax.axis_index("s")
    wid = s * 2 + c
    base = wid * _PER_W
    pltpu.sync_copy(idx_hbm.at[pl.ds(base, _PER_W)], idx_v)
    nch = _PER_W // _K

    def gather_slice(i, buf, gsem):
        return pltpu.make_async_copy(
            tab_hbm.at[idx_v.at[pl.ds(i * _K, _K)]], buf, gsem)

    def store_slice(i, buf, ssem):
        return pltpu.make_async_copy(
            buf, out_hbm.at[pl.ds(base + i * _K, _K)], ssem)

    gather_slice(0, buf0, g0).start()
    gather_slice(1, buf1, g1).start()

    @pl.loop(0, nch // 2)
    def _(ii):
        for par, (buf, gsem, ssem) in enumerate(
                ((buf0, g0, s0), (buf1, g1, s1))):
            i = ii * 2 + par
            gather_slice(i, buf, gsem).wait()
            st = store_slice(i, buf, ssem)
            st.start()

            @pl.when(i + 2 < nch)
            def _():
                st.wait()
                gather_slice(i + 2, buf, gsem).start()

    store_slice(nch - 2, buf0, s0).wait()
    store_slice(nch - 1, buf1, s1).wait()


def _sc_gather(table, idx):
    d = table.shape[1]
    mesh = plsc.VectorSubcoreMesh(core_axis_name="c", subcore_axis_name="s")
    fn = pl.kernel(
        _sc_gather_body,
        out_type=jax.ShapeDtypeStruct((_P, d), table.dtype),
        mesh=mesh,
        scratch_types=[
            pltpu.VMEM((_PER_W,), jnp.int32),
            pltpu.VMEM((_K, d), table.dtype),
            pltpu.VMEM((_K, d), table.dtype),
            pltpu.SemaphoreType.DMA,
            pltpu.SemaphoreType.DMA,
            pltpu.SemaphoreType.DMA,
            pltpu.SemaphoreType.DMA,
        ],
    )
    return fn(table, idx)


_KE = 512


def _sc_pair_body(a_hbm, b_hbm, idx_hbm, ao_hbm, bo_hbm,
                  idx_v, a0, a1, b0, b1, ga0, ga1, gb0, gb1,
                  sa0, sa1, sb0, sb1):
    c = lax.axis_index("c")
    s = lax.axis_index("s")
    wid = s * 2 + c
    base = wid * _PER_W
    pltpu.sync_copy(idx_hbm.at[pl.ds(base, _PER_W)], idx_v)
    nch = _PER_W // _KE

    def gathers(i, abuf, bbuf, gsa, gsb):
        sl = idx_v.at[pl.ds(i * _KE, _KE)]
        return (pltpu.make_async_copy(a_hbm.at[sl], abuf, gsa),
                pltpu.make_async_copy(b_hbm.at[sl], bbuf, gsb))

    def stores(i, abuf, bbuf, ssa, ssb):
        return (pltpu.make_async_copy(abuf, ao_hbm.at[pl.ds(base + i * _KE, _KE)], ssa),
                pltpu.make_async_copy(bbuf, bo_hbm.at[pl.ds(base + i * _KE, _KE)], ssb))

    for g in gathers(0, a0, b0, ga0, gb0):
        g.start()
    for g in gathers(1, a1, b1, ga1, gb1):
        g.start()

    @pl.loop(0, nch // 2)
    def _(ii):
        for par, (abuf, bbuf, gsa, gsb, ssa, ssb) in enumerate(
                ((a0, b0, ga0, gb0, sa0, sb0), (a1, b1, ga1, gb1, sa1, sb1))):
            i = ii * 2 + par
            for g in gathers(i, abuf, bbuf, gsa, gsb):
                g.wait()
            sts = stores(i, abuf, bbuf, ssa, ssb)
            for st in sts:
                st.start()

            @pl.when(i + 2 < nch)
            def _():
                for st in sts:
                    st.wait()
                for g in gathers(i + 2, abuf, bbuf, gsa, gsb):
                    g.start()

    for st in stores(nch - 2, a0, b0, sa0, sb0):
        st.wait()
    for st in stores(nch - 1, a1, b1, sa1, sb1):
        st.wait()


def _sc_pair_gather(a, b, idx):
    mesh = plsc.VectorSubcoreMesh(core_axis_name="c", subcore_axis_name="s")
    fn = pl.kernel(
        _sc_pair_body,
        out_type=(jax.ShapeDtypeStruct((_P,), a.dtype),
                  jax.ShapeDtypeStruct((_P,), b.dtype)),
        mesh=mesh,
        scratch_types=(
            [pltpu.VMEM((_PER_W,), jnp.int32)]
            + [pltpu.VMEM((_KE,), a.dtype) for _ in range(2)]
            + [pltpu.VMEM((_KE,), b.dtype) for _ in range(2)]
            + [pltpu.SemaphoreType.DMA for _ in range(8)]
        ),
    )
    return fn(a, b, idx)


def _sc_ea_body(p_hbm, q_hbm, row_hbm, col_hbm, out_hbm,
                p_v, q_v, r_v, c_v, ea_v):
    c = lax.axis_index("c")
    s = lax.axis_index("s")
    wid = s * 2 + c
    base = wid * _PER_W
    pltpu.sync_copy(p_hbm, p_v)
    pltpu.sync_copy(q_hbm, q_v)
    pltpu.sync_copy(row_hbm.at[pl.ds(base, _PER_W)], r_v)
    pltpu.sync_copy(col_hbm.at[pl.ds(base, _PER_W)], c_v)

    @pl.loop(0, _PER_W // 16)
    def _(i):
        ir = r_v[pl.ds(i * 16, 16)]
        ic = c_v[pl.ds(i * 16, 16)]
        pv = plsc.load_gather(p_v, [ir])
        qv = plsc.load_gather(q_v, [ic])
        ea_v[pl.ds(i * 16, 16)] = jnp.maximum(pv + qv, 0.0)

    pltpu.sync_copy(ea_v, out_hbm.at[pl.ds(base, _PER_W)])


def _sc_ea(p_ext, q_ext, row_s, col_s):
    mesh = plsc.VectorSubcoreMesh(core_axis_name="c", subcore_axis_name="s")
    fn = pl.kernel(
        _sc_ea_body,
        out_type=jax.ShapeDtypeStruct((_P,), jnp.float32),
        mesh=mesh,
        scratch_types=[
            pltpu.VMEM((_NEXT,), jnp.float32),
            pltpu.VMEM((_NEXT,), jnp.float32),
            pltpu.VMEM((_PER_W,), jnp.int32),
            pltpu.VMEM((_PER_W,), jnp.int32),
            pltpu.VMEM((_PER_W,), jnp.float32),
        ],
        compiler_params=pltpu.CompilerParams(needs_layout_passes=False),
    )
    return fn(p_ext, q_ext, row_s, col_s)


def _make_seg_kernel(edge_split):
    def _seg_kernel(c0_ref, m_ref, col_ref, val_ref, out_ref):
        h = pl.program_id(0)
        j = pl.program_id(1)

        @pl.when(j == 0)
        def _():
            out_ref[...] = jnp.zeros_like(out_ref)

        jj = j + h * pl.num_programs(1) if edge_split else j
        c0 = c0_ref[jj]
        loc = col_ref[0, 0, :] - c0 * 128
        vals = val_ref[0, 0, :]
        iota = lax.broadcasted_iota(jnp.int32, (_WIN, _EB), 0)
        onehot = (iota == loc[None, :]).astype(jnp.bfloat16)
        mv = m_ref[...] * vals[:, None]
        hi = mv.astype(jnp.bfloat16)
        lo = (mv - hi.astype(jnp.float32)).astype(jnp.bfloat16)
        prod = (jnp.dot(onehot, hi, preferred_element_type=jnp.float32)
                + jnp.dot(onehot, lo, preferred_element_type=jnp.float32))
        cur = out_ref[0, pl.ds(c0 * 128, _WIN), :]
        out_ref[0, pl.ds(c0 * 128, _WIN), :] = cur + prod

    return _seg_kernel


def _segment(c0, m, col3, val3, d):
    edge_split = True
    dh = d if edge_split else d // 2
    nj = _NB // 2 if edge_split else _NB
    if edge_split:
        m_map = lambda h, j, s: (h * nj + j, 0)
        e_map = lambda h, j, s: (h * nj + j, 0, 0)
    else:
        m_map = lambda h, j, s: (j, h)
        e_map = lambda h, j, s: (j, 0, 0)
    grid_spec = pltpu.PrefetchScalarGridSpec(
        num_scalar_prefetch=1,
        grid=(2, nj),
        in_specs=[
            pl.BlockSpec((_EB, dh), m_map),
            pl.BlockSpec((1, 1, _EB), e_map),
            pl.BlockSpec((1, 1, _EB), e_map),
        ],
        out_specs=pl.BlockSpec((1, N + 240, dh), lambda h, j, s: (h, 0, 0)),
    )
    accs = pl.pallas_call(
        _make_seg_kernel(edge_split),
        grid_spec=grid_spec,
        out_shape=jax.ShapeDtypeStruct((2, N + 240, dh), jnp.float32),
        compiler_params=pltpu.CompilerParams(
            dimension_semantics=("parallel", "arbitrary")),
    )(c0, m, col3, val3)
    if edge_split:
        return accs[0] + accs[1]
    return jnp.transpose(accs, (1, 0, 2)).reshape(N + 240, d)


def kernel(node_attr, edge_attr, edge_index, W1, b1, W2, b2, Wm, bm):
    row = edge_index[0]
    col = edge_index[1]
    ew = edge_attr.reshape(-1)

    npad = _P - E
    colp = jnp.concatenate([col, jnp.full((npad,), N - 1, jnp.int32)])
    rowp = jnp.concatenate(
        [row, N + (jnp.arange(npad, dtype=jnp.int32) % 16)])
    ewp = jnp.concatenate([ew, jnp.zeros((npad,), jnp.float32)])

    packed = (colp.astype(jnp.uint32) << 18) | jnp.arange(_P, dtype=jnp.uint32)
    packed_s = lax.sort(packed)
    col_s = (packed_s >> 18).astype(jnp.int32)
    id_s = (packed_s & 0x3FFFF).astype(jnp.int32)
    row_s, ew_s = _sc_pair_gather(rowp, ewp, id_s)

    c0 = col_s[::_EB] // 128
    col3 = col_s.reshape(_NB, 1, _EB)
    pad16 = ((0, 16), (0, 0))

    deg1 = jnp.ones((N,), jnp.float32).at[col_s].add(
        ew_s, indices_are_sorted=True)
    dinv1 = lax.rsqrt(deg1)
    xw1 = _matmul(node_attr, W1)
    z1 = xw1 * dinv1[:, None]
    m1 = _sc_gather(jnp.pad(z1, pad16), row_s)
    acc1 = _segment(c0, m1, col3, ew_s.reshape(_NB, 1, _EB), D_H)
    h = jax.nn.relu(dinv1[:, None] * (acc1[:N] + z1) + b1)

    wm_a = Wm[:D_H, :]
    wm_b = Wm[D_H:, :]
    wext = jnp.concatenate([W2, wm_a, wm_b], axis=1)
    wext = jnp.pad(wext, ((0, 0), (0, 126)))
    big = _matmul(h, wext)
    hw2 = big[:, :D_OUT]
    p = big[:, D_OUT]
    q = big[:, D_OUT + 1]

    p_ext = jnp.concatenate([p + bm[0], jnp.full((16,), -1e30, jnp.float32)])
    q_ext = jnp.concatenate([q, jnp.zeros((16,), jnp.float32)])
    ea_s = _sc_ea(p_ext, q_ext, row_s, col_s)

    deg2 = jnp.ones((N,), jnp.float32).at[col_s].add(
        ea_s, indices_are_sorted=True)
    dinv2 = lax.rsqrt(deg2)
    z2 = hw2 * dinv2[:, None]
    m2 = _sc_gather(jnp.pad(z2, pad16), row_s)
    acc2 = _segment(c0, m2, col3, ea_s.reshape(_NB, 1, _EB), D_OUT)
    out = dinv2[:, None] * (acc2[:N] + z2) + b2
    return out
